# baseline (speedup 1.0000x reference)
import functools

import jax
import jax.numpy as jnp
from jax import lax
from jax.experimental import pallas as pl
from jax.experimental.pallas import tpu as pltpu
import jax.experimental.pallas.tpu_sc as plsc

B = 32
D = 1024
K0, K1, K2 = 64, 128, 256
NP0 = 32 + K1 * 16
NP1 = 592 + K2 * 16
NP2 = 1168
R0 = NP0 // 16
R1 = NP1 // 16
R2 = NP2 // 16
_NC, _NS = 2, 16
_NBUF = 8
I_OUT = 370816


def _matmul_body(a_ref, w_ref, b_ref, o_ref):
    o_ref[...] = (
        jnp.dot(a_ref[...], w_ref[...], preferred_element_type=jnp.float32)
        + b_ref[...]
    )


def _matmul(A, W, bvec, BM=512, BN=512, interpret=False):
    M, K = A.shape
    N = W.shape[1]
    return pl.pallas_call(
        _matmul_body,
        grid=(M // BM, pl.cdiv(N, BN)),
        in_specs=[
            pl.BlockSpec((BM, K), lambda i, j: (i, 0)),
            pl.BlockSpec((K, BN), lambda i, j: (0, j)),
            pl.BlockSpec((1, BN), lambda i, j: (0, j)),
        ],
        out_specs=pl.BlockSpec((BM, BN), lambda i, j: (i, j)),
        out_shape=jax.ShapeDtypeStruct((M, N), jnp.float32),
        interpret=interpret,
    )(A, W, bvec.reshape(1, N))


def _sc_assemble(Y0, Y1, Y2, interpret=False):
    T0 = Y0.reshape(B * K0 * R0, 16)
    T1 = Y1.reshape(B * K1 * R1, 16)
    T2 = Y2.reshape(B * K2 * R2, 16)
    mesh = plsc.VectorSubcoreMesh(
        core_axis_name="c", subcore_axis_name="s", num_cores=_NC,
        num_subcores=_NS)

    @functools.partial(
        pl.kernel,
        mesh=mesh,
        out_type=jax.ShapeDtypeStruct((B, I_OUT), jnp.float32),
        scratch_types=[
            pltpu.VMEM((K0,), jnp.int32),
            pltpu.VMEM((K1,), jnp.int32),
            pltpu.VMEM((72 * 16,), jnp.int32),
            pltpu.VMEM((72 * 16,), jnp.int32),
            pltpu.VMEM((_NBUF, K0), jnp.int32),
            pltpu.VMEM((_NBUF, K1), jnp.int32),
            pltpu.VMEM((_NBUF, K0, 16), jnp.float32),
            pltpu.VMEM((_NBUF, K1, 16), jnp.float32),
            pltpu.VMEM((_NBUF, 37, 16), jnp.float32),
            pltpu.VMEM((_NBUF, 72, 16), jnp.float32),
            pltpu.VMEM((_NBUF, 576), jnp.float32),
            pltpu.VMEM((_NBUF, 1152), jnp.float32),
            pltpu.VMEM((K1, 16), jnp.float32),
            pltpu.VMEM((K1, 16), jnp.float32),
            pltpu.VMEM((1728,), jnp.float32),
            pltpu.VMEM((K2,), jnp.float32),
            pltpu.SemaphoreType.DMA((_NBUF,)),
            pltpu.SemaphoreType.DMA((_NBUF,)),
            pltpu.SemaphoreType.DMA((_NBUF,)),
            pltpu.SemaphoreType.DMA,
        ],
        compiler_params=pltpu.CompilerParams(use_tc_tiling_on_sc=False,
                                             needs_layout_passes=False),
        interpret=interpret,
    )
    def k(t0_hbm, t1_hbm, t2_hbm, y_hbm, base0, base1,
          rowtab, coltab, idx1, idx2, cb1, cb2, own1, own2, ob1, ob2,
          mg1, mg2, r0out, bout, gsem, hsem, wsem, msem):
        b = lax.axis_index("s") * _NC + lax.axis_index("c")
        lanes = lax.iota(jnp.int32, 16)
        for i in range(K0 // 16):
            base0[pl.ds(i * 16, 16)] = (b * K0 + i * 16 + lanes) * R0 + 2
        for i in range(K1 // 16):
            base1[pl.ds(i * 16, 16)] = (b * K1 + i * 16 + lanes) * R1 + 37

        def mktab(i, c):
            t = i * 16 + lanes
            q = t // 9
            rowtab[pl.ds(i * 16, 16)] = q
            coltab[pl.ds(i * 16, 16)] = t - q * 9
            return c
        lax.fori_loop(0, 72, mktab, 0, unroll=4)

        def blk1(blk, c):
            kn0 = blk * _NBUF
            for s in range(_NBUF):
                kn = kn0 + s
                for i in range(K0 // 16):
                    idx1[s, pl.ds(i * 16, 16)] = (
                        base0[pl.ds(i * 16, 16)] + kn)
                pltpu.async_copy(t0_hbm.at[idx1.at[s]], cb1.at[s],
                                 gsem.at[s])
                pltpu.async_copy(
                    t1_hbm.at[pl.ds((b * K1 + kn) * R1, 37), :],
                    own1.at[s], hsem.at[s])
            for s in range(_NBUF):
                kn = kn0 + s
                pltpu.make_async_copy(t0_hbm.at[idx1.at[s]], cb1.at[s],
                                      gsem.at[s]).wait()
                pltpu.make_async_copy(
                    t1_hbm.at[pl.ds((b * K1 + kn) * R1, 37), :],
                    own1.at[s], hsem.at[s]).wait()

                def inner(i, c2):
                    rv = rowtab[pl.ds(i * 16, 16)]
                    cv = coltab[pl.ds(i * 16, 16)]
                    g = plsc.load_gather(cb1.at[s], [rv, cv])
                    ob1[s, pl.ds(i * 16, 16)] = g + own1[s, i, :]
                    return c2
                lax.fori_loop(0, 36, inner, 0, unroll=4)
                pltpu.async_copy(
                    ob1.at[s], y_hbm.at[b, pl.ds(1792 + kn * 576, 576)],
                    wsem.at[s])
            for s in range(_NBUF):
                kn = kn0 + s
                pltpu.make_async_copy(
                    ob1.at[s], y_hbm.at[b, pl.ds(1792 + kn * 576, 576)],
                    wsem.at[s]).wait()
            return c
        lax.fori_loop(0, K1 // _NBUF, blk1, 0)

        def blk2(blk, c):
            kn0 = blk * _NBUF
            for s in range(_NBUF):
                kn = kn0 + s
                for i in range(K1 // 16):
                    idx2[s, pl.ds(i * 16, 16)] = (
                        base1[pl.ds(i * 16, 16)] + kn)
                pltpu.async_copy(t1_hbm.at[idx2.at[s]], cb2.at[s],
                                 gsem.at[s])
                pltpu.async_copy(
                    t2_hbm.at[pl.ds((b * K2 + kn) * R2, 72), :],
                    own2.at[s], hsem.at[s])
            for s in range(_NBUF):
                kn = kn0 + s
                pltpu.make_async_copy(t1_hbm.at[idx2.at[s]], cb2.at[s],
                                      gsem.at[s]).wait()
                pltpu.make_async_copy(
                    t2_hbm.at[pl.ds((b * K2 + kn) * R2, 72), :],
                    own2.at[s], hsem.at[s]).wait()

                def inner(i, c2):
                    rv = rowtab[pl.ds(i * 16, 16)]
                    cv = coltab[pl.ds(i * 16, 16)]
                    g = plsc.load_gather(cb2.at[s], [rv, cv])
                    ob2[s, pl.ds(i * 16, 16)] = g + own2[s, i, :]
                    return c2
                lax.fori_loop(0, 72, inner, 0, unroll=4)
                pltpu.async_copy(
                    ob2.at[s], y_hbm.at[b, pl.ds(75648 + kn * 1152, 1152)],
                    wsem.at[s])
            for s in range(_NBUF):
                kn = kn0 + s
                pltpu.make_async_copy(
                    ob2.at[s], y_hbm.at[b, pl.ds(75648 + kn * 1152, 1152)],
                    wsem.at[s]).wait()
            return c
        lax.fori_loop(0, K2 // _NBUF, blk2, 0)

        for i in range(K1 // 16):
            r = i * 16 + lanes
            idx2[0, pl.ds(i * 16, 16)] = (b * K0 + r // 2) * R0 + (r % 2)
        pltpu.async_copy(t0_hbm.at[idx2.at[0]], mg1, msem)
        pltpu.make_async_copy(t0_hbm.at[idx2.at[0]], mg1, msem).wait()

        def r0body(i, c):
            t = i * 16 + lanes
            q = t // 27
            f = 32 * q + (t - 27 * q)
            rv = f // 16
            g = plsc.load_gather(mg1, [rv, f - 16 * rv])
            r0out[pl.ds(i * 16, 16)] = g
            return c
        lax.fori_loop(0, 108, r0body, 0, unroll=4)
        pltpu.async_copy(r0out, y_hbm.at[b, pl.ds(0, 1728)], msem)
        pltpu.make_async_copy(r0out, y_hbm.at[b, pl.ds(0, 1728)],
                              msem).wait()

        def b0body(i, c):
            kk = i * 16 + lanes
            g = plsc.load_gather(mg1, [2 * kk + 1, lanes * 0 + 11])
            bout[pl.ds(i * 16, 16)] = g
            return c
        lax.fori_loop(0, 4, b0body, 0, unroll=4)
        pltpu.async_copy(bout.at[pl.ds(0, 64)],
                         y_hbm.at[b, pl.ds(1728, 64)], msem)
        pltpu.make_async_copy(bout.at[pl.ds(0, 64)],
                              y_hbm.at[b, pl.ds(1728, 64)], msem).wait()

        for i in range(K1 // 16):
            idx2[0, pl.ds(i * 16, 16)] = (
                (b * K1 + i * 16 + lanes) * R1 + 36)
        pltpu.async_copy(t1_hbm.at[idx2.at[0]], mg1, msem)
        pltpu.make_async_copy(t1_hbm.at[idx2.at[0]], mg1, msem).wait()

        def b1body(i, c):
            kk = i * 16 + lanes
            g = plsc.load_gather(mg1, [kk, lanes * 0])
            bout[pl.ds(i * 16, 16)] = g
            return c
        lax.fori_loop(0, 8, b1body, 0, unroll=4)
        pltpu.async_copy(bout.at[pl.ds(0, 128)],
                         y_hbm.at[b, pl.ds(75520, 128)], msem)
        pltpu.make_async_copy(bout.at[pl.ds(0, 128)],
                              y_hbm.at[b, pl.ds(75520, 128)], msem).wait()

        for half in range(2):
            for i in range(K1 // 16):
                idx2[0, pl.ds(i * 16, 16)] = (
                    (b * K2 + half * K1 + i * 16 + lanes) * R2 + 72)
            mg = mg1 if half == 0 else mg2
            pltpu.async_copy(t2_hbm.at[idx2.at[0]], mg, msem)
            pltpu.make_async_copy(t2_hbm.at[idx2.at[0]], mg, msem).wait()

            def b2body(i, c, mg=mg):
                kk = i * 16 + lanes
                g = plsc.load_gather(mg, [kk, lanes * 0])
                bout[pl.ds(i * 16, 16)] = g
                return c
            lax.fori_loop(0, 8, b2body, 0, unroll=4)
            pltpu.async_copy(
                bout.at[pl.ds(0, 128)],
                y_hbm.at[b, pl.ds(370560 + half * 128, 128)], msem)
            pltpu.make_async_copy(
                bout.at[pl.ds(0, 128)],
                y_hbm.at[b, pl.ds(370560 + half * 128, 128)], msem).wait()

    return k(T0, T1, T2)


def _forward(x, W0, b0, W1, b1, W2, b2, interpret=False):
    f32, bf16 = jnp.float32, jnp.bfloat16

    def pad_cross(Wc):
        nk = Wc.shape[1] // 9
        return jnp.pad(Wc.reshape(D, nk, 9), ((0, 0), (0, 0), (0, 7))
                       ).reshape(D, nk * 16)

    def pad_cross_b(bc):
        nk = bc.shape[0] // 9
        return jnp.pad(bc.reshape(nk, 9), ((0, 0), (0, 7))).reshape(nk * 16)

    W0p = jnp.concatenate(
        [W0[:, :28], jnp.zeros((D, 4), f32), pad_cross(0.5 * W0[:, 28:])],
        axis=1)
    b0p = jnp.concatenate(
        [b0[:28], jnp.zeros((4,), f32), pad_cross_b(0.5 * b0[28:])])
    W1p = jnp.concatenate(
        [0.5 * W1[:, :577], jnp.zeros((D, 15), f32),
         pad_cross(W1[:, 577:])], axis=1)
    b1p = jnp.concatenate(
        [0.5 * b1[:577], jnp.zeros((15,), f32), pad_cross_b(b1[577:])])
    W2p = jnp.pad(W2, ((0, 0), (0, NP2 - 1153)))
    b2p = jnp.pad(b2, ((0, NP2 - 1153),))

    xb = x.astype(bf16)
    X0 = xb[:, :K0].reshape(B * K0, D)
    X1 = xb[:, K0:K0 + K1].reshape(B * K1, D)
    X2 = xb[:, K0 + K1:].reshape(B * K2, D)

    Y0 = _matmul(X0, W0p.astype(bf16), b0p, interpret=interpret)
    Y1 = _matmul(X1, W1p.astype(bf16), b1p, interpret=interpret)
    Y2 = _matmul(X2, W2p.astype(bf16), b2p, interpret=interpret)

    return _sc_assemble(Y0, Y1, Y2, interpret=interpret)


def kernel(x, W0, b0, W1, b1, W2, b2):
    return _forward(x, W0, b0, W1, b1, W2, b2)


# speedup vs baseline: 1.1073x; 1.1073x over previous
import functools

import jax
import jax.numpy as jnp
from jax import lax
from jax.experimental import pallas as pl
from jax.experimental.pallas import tpu as pltpu
import jax.experimental.pallas.tpu_sc as plsc

B = 32
D = 1024
K0, K1, K2 = 64, 128, 256
NP0 = 32 + K1 * 16
NP1 = 592 + K2 * 16
NP2 = 1168
R0 = NP0 // 16
R1 = NP1 // 16
R2 = NP2 // 16
_NC, _NS = 2, 16
_NBUF = 8
I_OUT = 370816


def _matmul_body(a_ref, w_ref, b_ref, o_ref):
    o_ref[...] = (
        jnp.dot(a_ref[...], w_ref[...], preferred_element_type=jnp.float32)
        + b_ref[...]
    )


def _matmul(A, W, bvec, BM=256, interpret=False):
    M, K = A.shape
    N = W.shape[1]
    return pl.pallas_call(
        _matmul_body,
        grid=(M // BM,),
        in_specs=[
            pl.BlockSpec((BM, K), lambda i: (i, 0)),
            pl.BlockSpec((K, N), lambda i: (0, 0)),
            pl.BlockSpec((1, N), lambda i: (0, 0)),
        ],
        out_specs=pl.BlockSpec((BM, N), lambda i: (i, 0)),
        out_shape=jax.ShapeDtypeStruct((M, N), jnp.float32),
        interpret=interpret,
    )(A, W, bvec.reshape(1, N))


def _sc_assemble(Y0, Y1, Y2, interpret=False):
    T0 = Y0.reshape(B * K0 * R0, 16)
    T1 = Y1.reshape(B * K1 * R1, 16)
    T2 = Y2.reshape(B * K2 * R2, 16)
    mesh = plsc.VectorSubcoreMesh(
        core_axis_name="c", subcore_axis_name="s", num_cores=_NC,
        num_subcores=_NS)

    @functools.partial(
        pl.kernel,
        mesh=mesh,
        out_type=jax.ShapeDtypeStruct((B, I_OUT), jnp.float32),
        scratch_types=[
            pltpu.VMEM((K0,), jnp.int32),
            pltpu.VMEM((K1,), jnp.int32),
            pltpu.VMEM((72 * 16,), jnp.int32),
            pltpu.VMEM((72 * 16,), jnp.int32),
            pltpu.VMEM((_NBUF, K0), jnp.int32),
            pltpu.VMEM((_NBUF, K1), jnp.int32),
            pltpu.VMEM((_NBUF, K0, 16), jnp.float32),
            pltpu.VMEM((_NBUF, K1, 16), jnp.float32),
            pltpu.VMEM((_NBUF, 37, 16), jnp.float32),
            pltpu.VMEM((_NBUF, 72, 16), jnp.float32),
            pltpu.VMEM((_NBUF, 576), jnp.float32),
            pltpu.VMEM((_NBUF, 1152), jnp.float32),
            pltpu.VMEM((K1, 16), jnp.float32),
            pltpu.VMEM((K1, 16), jnp.float32),
            pltpu.VMEM((1728,), jnp.float32),
            pltpu.VMEM((K2,), jnp.float32),
            pltpu.SemaphoreType.DMA((_NBUF,)),
            pltpu.SemaphoreType.DMA((_NBUF,)),
            pltpu.SemaphoreType.DMA((_NBUF,)),
            pltpu.SemaphoreType.DMA,
        ],
        compiler_params=pltpu.CompilerParams(use_tc_tiling_on_sc=False,
                                             needs_layout_passes=False),
        interpret=interpret,
    )
    def k(t0_hbm, t1_hbm, t2_hbm, y_hbm, base0, base1,
          rowtab, coltab, idx1, idx2, cb1, cb2, own1, own2, ob1, ob2,
          mg1, mg2, r0out, bout, gsem, hsem, wsem, msem):
        b = lax.axis_index("s") * _NC + lax.axis_index("c")
        lanes = lax.iota(jnp.int32, 16)
        for i in range(K0 // 16):
            base0[pl.ds(i * 16, 16)] = (b * K0 + i * 16 + lanes) * R0 + 2
        for i in range(K1 // 16):
            base1[pl.ds(i * 16, 16)] = (b * K1 + i * 16 + lanes) * R1 + 37

        def mktab(i, c):
            t = i * 16 + lanes
            q = t // 9
            rowtab[pl.ds(i * 16, 16)] = q
            coltab[pl.ds(i * 16, 16)] = t - q * 9
            return c
        lax.fori_loop(0, 72, mktab, 0, unroll=4)

        def blk1(blk, c):
            kn0 = blk * _NBUF
            for s in range(_NBUF):
                kn = kn0 + s
                for i in range(K0 // 16):
                    idx1[s, pl.ds(i * 16, 16)] = (
                        base0[pl.ds(i * 16, 16)] + kn)
                pltpu.async_copy(t0_hbm.at[idx1.at[s]], cb1.at[s],
                                 gsem.at[s])
                pltpu.async_copy(
                    t1_hbm.at[pl.ds((b * K1 + kn) * R1, 37), :],
                    own1.at[s], hsem.at[s])
            for s in range(_NBUF):
                kn = kn0 + s
                pltpu.make_async_copy(t0_hbm.at[idx1.at[s]], cb1.at[s],
                                      gsem.at[s]).wait()
                pltpu.make_async_copy(
                    t1_hbm.at[pl.ds((b * K1 + kn) * R1, 37), :],
                    own1.at[s], hsem.at[s]).wait()

                def inner(i, c2):
                    rv = rowtab[pl.ds(i * 16, 16)]
                    cv = coltab[pl.ds(i * 16, 16)]
                    g = plsc.load_gather(cb1.at[s], [rv, cv])
                    ob1[s, pl.ds(i * 16, 16)] = g + own1[s, i, :]
                    return c2
                lax.fori_loop(0, 36, inner, 0, unroll=4)
                pltpu.async_copy(
                    ob1.at[s], y_hbm.at[b, pl.ds(1792 + kn * 576, 576)],
                    wsem.at[s])
            for s in range(_NBUF):
                kn = kn0 + s
                pltpu.make_async_copy(
                    ob1.at[s], y_hbm.at[b, pl.ds(1792 + kn * 576, 576)],
                    wsem.at[s]).wait()
            return c
        lax.fori_loop(0, K1 // _NBUF, blk1, 0)

        def blk2(blk, c):
            kn0 = blk * _NBUF
            for s in range(_NBUF):
                kn = kn0 + s
                for i in range(K1 // 16):
                    idx2[s, pl.ds(i * 16, 16)] = (
                        base1[pl.ds(i * 16, 16)] + kn)
                pltpu.async_copy(t1_hbm.at[idx2.at[s]], cb2.at[s],
                                 gsem.at[s])
                pltpu.async_copy(
                    t2_hbm.at[pl.ds((b * K2 + kn) * R2, 72), :],
                    own2.at[s], hsem.at[s])
            for s in range(_NBUF):
                kn = kn0 + s
                pltpu.make_async_copy(t1_hbm.at[idx2.at[s]], cb2.at[s],
                                      gsem.at[s]).wait()
                pltpu.make_async_copy(
                    t2_hbm.at[pl.ds((b * K2 + kn) * R2, 72), :],
                    own2.at[s], hsem.at[s]).wait()

                def inner(i, c2):
                    rv = rowtab[pl.ds(i * 16, 16)]
                    cv = coltab[pl.ds(i * 16, 16)]
                    g = plsc.load_gather(cb2.at[s], [rv, cv])
                    ob2[s, pl.ds(i * 16, 16)] = g + own2[s, i, :]
                    return c2
                lax.fori_loop(0, 72, inner, 0, unroll=4)
                pltpu.async_copy(
                    ob2.at[s], y_hbm.at[b, pl.ds(75648 + kn * 1152, 1152)],
                    wsem.at[s])
            for s in range(_NBUF):
                kn = kn0 + s
                pltpu.make_async_copy(
                    ob2.at[s], y_hbm.at[b, pl.ds(75648 + kn * 1152, 1152)],
                    wsem.at[s]).wait()
            return c
        lax.fori_loop(0, K2 // _NBUF, blk2, 0)

        for i in range(K1 // 16):
            r = i * 16 + lanes
            idx2[0, pl.ds(i * 16, 16)] = (b * K0 + r // 2) * R0 + (r % 2)
        pltpu.async_copy(t0_hbm.at[idx2.at[0]], mg1, msem)
        pltpu.make_async_copy(t0_hbm.at[idx2.at[0]], mg1, msem).wait()

        def r0body(i, c):
            t = i * 16 + lanes
            q = t // 27
            f = 32 * q + (t - 27 * q)
            rv = f // 16
            g = plsc.load_gather(mg1, [rv, f - 16 * rv])
            r0out[pl.ds(i * 16, 16)] = g
            return c
        lax.fori_loop(0, 108, r0body, 0, unroll=4)
        pltpu.async_copy(r0out, y_hbm.at[b, pl.ds(0, 1728)], msem)
        pltpu.make_async_copy(r0out, y_hbm.at[b, pl.ds(0, 1728)],
                              msem).wait()

        def b0body(i, c):
            kk = i * 16 + lanes
            g = plsc.load_gather(mg1, [2 * kk + 1, lanes * 0 + 11])
            bout[pl.ds(i * 16, 16)] = g
            return c
        lax.fori_loop(0, 4, b0body, 0, unroll=4)
        pltpu.async_copy(bout.at[pl.ds(0, 64)],
                         y_hbm.at[b, pl.ds(1728, 64)], msem)
        pltpu.make_async_copy(bout.at[pl.ds(0, 64)],
                              y_hbm.at[b, pl.ds(1728, 64)], msem).wait()

        for i in range(K1 // 16):
            idx2[0, pl.ds(i * 16, 16)] = (
                (b * K1 + i * 16 + lanes) * R1 + 36)
        pltpu.async_copy(t1_hbm.at[idx2.at[0]], mg1, msem)
        pltpu.make_async_copy(t1_hbm.at[idx2.at[0]], mg1, msem).wait()

        def b1body(i, c):
            kk = i * 16 + lanes
            g = plsc.load_gather(mg1, [kk, lanes * 0])
            bout[pl.ds(i * 16, 16)] = g
            return c
        lax.fori_loop(0, 8, b1body, 0, unroll=4)
        pltpu.async_copy(bout.at[pl.ds(0, 128)],
                         y_hbm.at[b, pl.ds(75520, 128)], msem)
        pltpu.make_async_copy(bout.at[pl.ds(0, 128)],
                              y_hbm.at[b, pl.ds(75520, 128)], msem).wait()

        for half in range(2):
            for i in range(K1 // 16):
                idx2[0, pl.ds(i * 16, 16)] = (
                    (b * K2 + half * K1 + i * 16 + lanes) * R2 + 72)
            mg = mg1 if half == 0 else mg2
            pltpu.async_copy(t2_hbm.at[idx2.at[0]], mg, msem)
            pltpu.make_async_copy(t2_hbm.at[idx2.at[0]], mg, msem).wait()

            def b2body(i, c, mg=mg):
                kk = i * 16 + lanes
                g = plsc.load_gather(mg, [kk, lanes * 0])
                bout[pl.ds(i * 16, 16)] = g
                return c
            lax.fori_loop(0, 8, b2body, 0, unroll=4)
            pltpu.async_copy(
                bout.at[pl.ds(0, 128)],
                y_hbm.at[b, pl.ds(370560 + half * 128, 128)], msem)
            pltpu.make_async_copy(
                bout.at[pl.ds(0, 128)],
                y_hbm.at[b, pl.ds(370560 + half * 128, 128)], msem).wait()

    return k(T0, T1, T2)


def _forward(x, W0, b0, W1, b1, W2, b2, interpret=False):
    f32, bf16 = jnp.float32, jnp.bfloat16

    def pad_cross(Wc):
        nk = Wc.shape[1] // 9
        return jnp.pad(Wc.reshape(D, nk, 9), ((0, 0), (0, 0), (0, 7))
                       ).reshape(D, nk * 16)

    def pad_cross_b(bc):
        nk = bc.shape[0] // 9
        return jnp.pad(bc.reshape(nk, 9), ((0, 0), (0, 7))).reshape(nk * 16)

    W0p = jnp.concatenate(
        [W0[:, :28], jnp.zeros((D, 4), f32), pad_cross(0.5 * W0[:, 28:])],
        axis=1)
    b0p = jnp.concatenate(
        [b0[:28], jnp.zeros((4,), f32), pad_cross_b(0.5 * b0[28:])])
    W1p = jnp.concatenate(
        [0.5 * W1[:, :577], jnp.zeros((D, 15), f32),
         pad_cross(W1[:, 577:])], axis=1)
    b1p = jnp.concatenate(
        [0.5 * b1[:577], jnp.zeros((15,), f32), pad_cross_b(b1[577:])])
    W2p = jnp.pad(W2, ((0, 0), (0, NP2 - 1153)))
    b2p = jnp.pad(b2, ((0, NP2 - 1153),))

    xb = x.astype(bf16)
    X0 = xb[:, :K0].reshape(B * K0, D)
    X1 = xb[:, K0:K0 + K1].reshape(B * K1, D)
    X2 = xb[:, K0 + K1:].reshape(B * K2, D)

    Y0 = _matmul(X0, W0p.astype(bf16), b0p, interpret=interpret)
    Y1 = _matmul(X1, W1p.astype(bf16), b1p, interpret=interpret)
    Y2 = _matmul(X2, W2p.astype(bf16), b2p, interpret=interpret)

    return _sc_assemble(Y0, Y1, Y2, interpret=interpret)


def kernel(x, W0, b0, W1, b1, W2, b2):
    return _forward(x, W0, b0, W1, b1, W2, b2)


# speedup vs baseline: 1.1934x; 1.0777x over previous
import functools

import jax
import jax.numpy as jnp
from jax import lax
from jax.experimental import pallas as pl
from jax.experimental.pallas import tpu as pltpu
import jax.experimental.pallas.tpu_sc as plsc

B = 32
D = 1024
K0, K1, K2 = 64, 128, 256
NP0 = 32 + K1 * 16
NP1 = 592 + K2 * 16
NP2 = 1168
R0 = NP0 // 16
R1 = NP1 // 16
R2 = NP2 // 16
_NC, _NS = 2, 16
_NBUF = 8
I_OUT = 370816


def _matmul_body(a_ref, w_ref, b_ref, o_ref):
    o_ref[...] = (
        jnp.dot(a_ref[...], w_ref[...], preferred_element_type=jnp.float32)
        + b_ref[...]
    )


def _matmul(A, W, bvec, BM=256, interpret=False):
    M, K = A.shape
    N = W.shape[1]
    return pl.pallas_call(
        _matmul_body,
        grid=(M // BM,),
        in_specs=[
            pl.BlockSpec((BM, K), lambda i: (i, 0)),
            pl.BlockSpec((K, N), lambda i: (0, 0)),
            pl.BlockSpec((1, N), lambda i: (0, 0)),
        ],
        out_specs=pl.BlockSpec((BM, N), lambda i: (i, 0)),
        out_shape=jax.ShapeDtypeStruct((M, N), jnp.float32),
        interpret=interpret,
    )(A, W, bvec.reshape(1, N))


def _sc_assemble(Y0, Y1, Y2, interpret=False):
    T0 = Y0.reshape(B * K0 * R0, 16)
    T1 = Y1.reshape(B * K1 * R1, 16)
    T2 = Y2.reshape(B * K2 * R2, 16)
    mesh = plsc.VectorSubcoreMesh(
        core_axis_name="c", subcore_axis_name="s", num_cores=_NC,
        num_subcores=_NS)

    @functools.partial(
        pl.kernel,
        mesh=mesh,
        out_type=jax.ShapeDtypeStruct((B, I_OUT), jnp.float32),
        scratch_types=[
            pltpu.VMEM((K0,), jnp.int32),
            pltpu.VMEM((K1,), jnp.int32),
            pltpu.VMEM((72 * 16,), jnp.int32),
            pltpu.VMEM((72 * 16,), jnp.int32),
            pltpu.VMEM((_NBUF, K0), jnp.int32),
            pltpu.VMEM((_NBUF, K1), jnp.int32),
            pltpu.VMEM((_NBUF, K0, 16), jnp.float32),
            pltpu.VMEM((_NBUF, K1, 16), jnp.float32),
            pltpu.VMEM((_NBUF, 37, 16), jnp.float32),
            pltpu.VMEM((_NBUF, 72, 16), jnp.float32),
            pltpu.VMEM((_NBUF, 576), jnp.float32),
            pltpu.VMEM((_NBUF, 1152), jnp.float32),
            pltpu.VMEM((K1, 16), jnp.float32),
            pltpu.VMEM((K1, 16), jnp.float32),
            pltpu.VMEM((1728,), jnp.float32),
            pltpu.VMEM((K2,), jnp.float32),
            pltpu.SemaphoreType.DMA((_NBUF,)),
            pltpu.SemaphoreType.DMA((_NBUF,)),
            pltpu.SemaphoreType.DMA((_NBUF,)),
            pltpu.SemaphoreType.DMA,
        ],
        compiler_params=pltpu.CompilerParams(use_tc_tiling_on_sc=False,
                                             needs_layout_passes=False),
        interpret=interpret,
    )
    def k(t0_hbm, t1_hbm, t2_hbm, y_hbm, base0, base1,
          rowtab, coltab, idx1, idx2, cb1, cb2, own1, own2, ob1, ob2,
          mg1, mg2, r0out, bout, gsem, hsem, wsem, msem):
        b = lax.axis_index("s") * _NC + lax.axis_index("c")
        lanes = lax.iota(jnp.int32, 16)
        for i in range(K0 // 16):
            base0[pl.ds(i * 16, 16)] = (b * K0 + i * 16 + lanes) * R0 + 2
        for i in range(K1 // 16):
            base1[pl.ds(i * 16, 16)] = (b * K1 + i * 16 + lanes) * R1 + 37

        def mktab(i, c):
            t = i * 16 + lanes
            q = t // 9
            rowtab[pl.ds(i * 16, 16)] = q
            coltab[pl.ds(i * 16, 16)] = t - q * 9
            return c
        lax.fori_loop(0, 72, mktab, 0, unroll=4)

        def run_region(nunits, nvec, base, idx, cb, own, ob, nrows, t_hbm,
                       tab_hbm, rpk, niter, ybase, width):
            def fill(s, kn):
                for i in range(nvec):
                    idx[s, pl.ds(i * 16, 16)] = base[pl.ds(i * 16, 16)] + kn
                pltpu.async_copy(t_hbm.at[idx.at[s]], cb.at[s], gsem.at[s])
                pltpu.async_copy(
                    tab_hbm.at[pl.ds((b * nunits + kn) * rpk, nrows), :],
                    own.at[s], hsem.at[s])

            for s in range(_NBUF):
                fill(s, s)

            def blk(blk_i, c):
                kn0 = blk_i * _NBUF
                for s in range(_NBUF):
                    kn = kn0 + s
                    pltpu.make_async_copy(t_hbm.at[idx.at[s]], cb.at[s],
                                          gsem.at[s]).wait()
                    pltpu.make_async_copy(
                        tab_hbm.at[pl.ds((b * nunits + kn) * rpk, nrows),
                                   :],
                        own.at[s], hsem.at[s]).wait()

                    @pl.when(blk_i > 0)
                    def _():
                        pltpu.make_async_copy(
                            ob.at[s],
                            y_hbm.at[b, pl.ds(ybase + (kn - _NBUF) * width,
                                              width)],
                            wsem.at[s]).wait()

                    def inner(i, c2):
                        rv = rowtab[pl.ds(i * 16, 16)]
                        cv = coltab[pl.ds(i * 16, 16)]
                        g = plsc.load_gather(cb.at[s], [rv, cv])
                        ob[s, pl.ds(i * 16, 16)] = g + own[s, i, :]
                        return c2
                    lax.fori_loop(0, niter, inner, 0, unroll=4)
                    pltpu.async_copy(
                        ob.at[s], y_hbm.at[b, pl.ds(ybase + kn * width,
                                                    width)],
                        wsem.at[s])

                    @pl.when(blk_i < nunits // _NBUF - 1)
                    def _():
                        fill(s, kn + _NBUF)
                return c
            lax.fori_loop(0, nunits // _NBUF, blk, 0)
            for s in range(_NBUF):
                kn = nunits - _NBUF + s
                pltpu.make_async_copy(
                    ob.at[s], y_hbm.at[b, pl.ds(ybase + kn * width, width)],
                    wsem.at[s]).wait()

        run_region(K1, K0 // 16, base0, idx1, cb1, own1, ob1, 37, t0_hbm,
                   t1_hbm, R1, 36, 1792, 576)
        run_region(K2, K1 // 16, base1, idx2, cb2, own2, ob2, 72, t1_hbm,
                   t2_hbm, R2, 72, 75648, 1152)

        for i in range(K1 // 16):
            r = i * 16 + lanes
            idx2[0, pl.ds(i * 16, 16)] = (b * K0 + r // 2) * R0 + (r % 2)
        pltpu.async_copy(t0_hbm.at[idx2.at[0]], mg1, msem)
        pltpu.make_async_copy(t0_hbm.at[idx2.at[0]], mg1, msem).wait()

        def r0body(i, c):
            t = i * 16 + lanes
            q = t // 27
            f = 32 * q + (t - 27 * q)
            rv = f // 16
            g = plsc.load_gather(mg1, [rv, f - 16 * rv])
            r0out[pl.ds(i * 16, 16)] = g
            return c
        lax.fori_loop(0, 108, r0body, 0, unroll=4)
        pltpu.async_copy(r0out, y_hbm.at[b, pl.ds(0, 1728)], msem)
        pltpu.make_async_copy(r0out, y_hbm.at[b, pl.ds(0, 1728)],
                              msem).wait()

        def b0body(i, c):
            kk = i * 16 + lanes
            g = plsc.load_gather(mg1, [2 * kk + 1, lanes * 0 + 11])
            bout[pl.ds(i * 16, 16)] = g
            return c
        lax.fori_loop(0, 4, b0body, 0, unroll=4)
        pltpu.async_copy(bout.at[pl.ds(0, 64)],
                         y_hbm.at[b, pl.ds(1728, 64)], msem)
        pltpu.make_async_copy(bout.at[pl.ds(0, 64)],
                              y_hbm.at[b, pl.ds(1728, 64)], msem).wait()

        for i in range(K1 // 16):
            idx2[0, pl.ds(i * 16, 16)] = (
                (b * K1 + i * 16 + lanes) * R1 + 36)
        pltpu.async_copy(t1_hbm.at[idx2.at[0]], mg1, msem)
        pltpu.make_async_copy(t1_hbm.at[idx2.at[0]], mg1, msem).wait()

        def b1body(i, c):
            kk = i * 16 + lanes
            g = plsc.load_gather(mg1, [kk, lanes * 0])
            bout[pl.ds(i * 16, 16)] = g
            return c
        lax.fori_loop(0, 8, b1body, 0, unroll=4)
        pltpu.async_copy(bout.at[pl.ds(0, 128)],
                         y_hbm.at[b, pl.ds(75520, 128)], msem)
        pltpu.make_async_copy(bout.at[pl.ds(0, 128)],
                              y_hbm.at[b, pl.ds(75520, 128)], msem).wait()

        for half in range(2):
            for i in range(K1 // 16):
                idx2[0, pl.ds(i * 16, 16)] = (
                    (b * K2 + half * K1 + i * 16 + lanes) * R2 + 72)
            mg = mg1 if half == 0 else mg2
            pltpu.async_copy(t2_hbm.at[idx2.at[0]], mg, msem)
            pltpu.make_async_copy(t2_hbm.at[idx2.at[0]], mg, msem).wait()

            def b2body(i, c, mg=mg):
                kk = i * 16 + lanes
                g = plsc.load_gather(mg, [kk, lanes * 0])
                bout[pl.ds(i * 16, 16)] = g
                return c
            lax.fori_loop(0, 8, b2body, 0, unroll=4)
            pltpu.async_copy(
                bout.at[pl.ds(0, 128)],
                y_hbm.at[b, pl.ds(370560 + half * 128, 128)], msem)
            pltpu.make_async_copy(
                bout.at[pl.ds(0, 128)],
                y_hbm.at[b, pl.ds(370560 + half * 128, 128)], msem).wait()

    return k(T0, T1, T2)


def _forward(x, W0, b0, W1, b1, W2, b2, interpret=False):
    f32, bf16 = jnp.float32, jnp.bfloat16

    def pad_cross(Wc):
        nk = Wc.shape[1] // 9
        return jnp.pad(Wc.reshape(D, nk, 9), ((0, 0), (0, 0), (0, 7))
                       ).reshape(D, nk * 16)

    def pad_cross_b(bc):
        nk = bc.shape[0] // 9
        return jnp.pad(bc.reshape(nk, 9), ((0, 0), (0, 7))).reshape(nk * 16)

    W0p = jnp.concatenate(
        [W0[:, :28], jnp.zeros((D, 4), f32), pad_cross(0.5 * W0[:, 28:])],
        axis=1)
    b0p = jnp.concatenate(
        [b0[:28], jnp.zeros((4,), f32), pad_cross_b(0.5 * b0[28:])])
    W1p = jnp.concatenate(
        [0.5 * W1[:, :577], jnp.zeros((D, 15), f32),
         pad_cross(W1[:, 577:])], axis=1)
    b1p = jnp.concatenate(
        [0.5 * b1[:577], jnp.zeros((15,), f32), pad_cross_b(b1[577:])])
    W2p = jnp.pad(W2, ((0, 0), (0, NP2 - 1153)))
    b2p = jnp.pad(b2, ((0, NP2 - 1153),))

    xb = x.astype(bf16)
    X0 = xb[:, :K0].reshape(B * K0, D)
    X1 = xb[:, K0:K0 + K1].reshape(B * K1, D)
    X2 = xb[:, K0 + K1:].reshape(B * K2, D)

    Y0 = _matmul(X0, W0p.astype(bf16), b0p, interpret=interpret)
    Y1 = _matmul(X1, W1p.astype(bf16), b1p, interpret=interpret)
    Y2 = _matmul(X2, W2p.astype(bf16), b2p, interpret=interpret)

    return _sc_assemble(Y0, Y1, Y2, interpret=interpret)


def kernel(x, W0, b0, W1, b1, W2, b2):
    return _forward(x, W0, b0, W1, b1, W2, b2)
